# baseline (speedup 1.0000x reference)
import jax
import jax.numpy as jnp
from jax import lax
from jax.experimental import pallas as pl
from jax.experimental.pallas import tpu as pltpu
from jax.experimental.pallas import tpu_sc as plsc

N = 65536
C = 100
NBINS = 15
SHIFT = 17
NBUCKETS = 8192
NBSC = 8128

NBA = 1024


def _softmax_body(logits_ref, labels_ref, conft_ref, idxt_ref, c2t_ref,
                  yt_ref, cyt_ref, minv_ref):
    i = pl.program_id(0)
    x = logits_ref[...]
    m = jnp.max(x, axis=1, keepdims=True)
    e = jnp.exp(x - m)
    s = jnp.sum(e, axis=1, keepdims=True)
    conf = e / s
    confp = jnp.pad(conf, ((0, 0), (0, 128 - C)))
    conft = confp.T[0:C, :]
    conft_ref[...] = conft
    bits = lax.bitcast_convert_type(conft, jnp.int32)
    bucket = lax.shift_right_logical(bits, SHIFT)
    bucket = jnp.minimum(bucket, NBSC - 1)
    rows = lax.broadcasted_iota(jnp.int32, (C, 1), 0)
    local = jnp.mod(rows, 25)
    idxt_ref[...] = bucket + local * NBSC
    c2t_ref[...] = conft * conft
    labels = labels_ref[0]
    y = (labels == rows).astype(jnp.float32)
    yt_ref[...] = y
    cyt_ref[...] = conft * y
    mn = jnp.min(conft, axis=1, keepdims=True)

    @pl.when(i == 0)
    def _():
        minv_ref[...] = mn

    @pl.when(i > 0)
    def _():
        minv_ref[...] = jnp.minimum(minv_ref[...], mn)


def _softmax_call(logits, labels3d):
    big = jax.ShapeDtypeStruct((C, N), jnp.float32)
    return pl.pallas_call(
        _softmax_body,
        grid=(N // NBA,),
        in_specs=[
            pl.BlockSpec((NBA, C), lambda i: (i, 0)),
            pl.BlockSpec((1, 1, NBA), lambda i: (i, 0, 0)),
        ],
        out_specs=[
            pl.BlockSpec((C, NBA), lambda i: (0, i)),
            pl.BlockSpec((C, NBA), lambda i: (0, i)),
            pl.BlockSpec((C, NBA), lambda i: (0, i)),
            pl.BlockSpec((C, NBA), lambda i: (0, i)),
            pl.BlockSpec((C, NBA), lambda i: (0, i)),
            pl.BlockSpec((C, 1), lambda i: (0, 0)),
        ],
        out_shape=[
            big,
            jax.ShapeDtypeStruct((C, N), jnp.int32),
            big, big, big,
            jax.ShapeDtypeStruct((C, 1), jnp.float32),
        ],
    )(logits, labels3d)


NCORES = 2
CLS_PER_CORE = C // NCORES
CH = 8192
NCHUNK = N // CH


def _make_hist_body(base):
    def _hist_body(conf_hbm, idx_hbm, c2_hbm, y_hbm, cy_hbm, ones_hbm,
                   zeros_hbm, out_hbm, idx_v, vc_v, vc2_v, vy_v, vcy_v,
                   ones_v, stage_v, sem, h1_sh, hc_sh, hc2_sh, hy_sh,
                   hcy_sh):
        cid = lax.axis_index("c")
        sid = lax.axis_index("s")
        cnt = jnp.where(sid < 9, 2, 1)
        start_local = sid + jnp.minimum(sid, 9)
        shs = (h1_sh, hc_sh, hc2_sh, hy_sh, hcy_sh)

        pltpu.sync_copy(ones_hbm, ones_v)
        pltpu.sync_copy(zeros_hbm, stage_v)

        def do_class(z):
            local = start_local + z
            cls = cid * CLS_PER_CORE + base + local
            hbase = local * NBSC
            for sh in shs:
                pltpu.sync_copy(stage_v, sh.at[pl.ds(hbase, NBSC)])
            for ch in range(NCHUNK):
                sl = pl.ds(ch * CH, CH)
                descs = [
                    pltpu.async_copy(idx_hbm.at[cls, sl], idx_v, sem),
                    pltpu.async_copy(conf_hbm.at[cls, sl], vc_v, sem),
                    pltpu.async_copy(c2_hbm.at[cls, sl], vc2_v, sem),
                    pltpu.async_copy(y_hbm.at[cls, sl], vy_v, sem),
                    pltpu.async_copy(cy_hbm.at[cls, sl], vcy_v, sem),
                ]
                for d in descs:
                    d.wait()
                for sh, val in zip(shs, (ones_v, vc_v, vc2_v, vy_v, vcy_v)):
                    pltpu.sync_copy(val, sh.at[plsc.Indices(idx_v)],
                                    add=True)
            for q, sh in enumerate(shs):
                pltpu.sync_copy(sh.at[pl.ds(hbase, NBSC)], stage_v)
                pltpu.sync_copy(stage_v, out_hbm.at[(q * 2 + cid) * 25
                                                    + local])
            pltpu.sync_copy(zeros_hbm, stage_v)

        def loop_body(z, _):
            @pl.when(z < cnt)
            def _():
                do_class(z)
            return 0

        lax.fori_loop(0, 2, loop_body, 0)

    return _hist_body


def _hist_call(conft, idxt, c2t, yt, cyt):
    mesh = plsc.VectorSubcoreMesh(core_axis_name="c", subcore_axis_name="s")
    ones = jnp.ones((CH,), jnp.float32)
    zeros = jnp.zeros((NBSC,), jnp.float32)
    buf = pltpu.VMEM((CH,), jnp.float32)
    shared = pltpu.VMEM_SHARED((25 * NBSC,), jnp.float32)
    halves = []
    for base in (0, 25):
        fn = pl.kernel(
            _make_hist_body(base),
            out_type=jax.ShapeDtypeStruct((5 * 2 * 25, NBSC), jnp.float32),
            mesh=mesh,
            scratch_types=[
                pltpu.VMEM((CH,), jnp.int32),
                buf, buf, buf, buf,
                buf,
                pltpu.VMEM((NBSC,), jnp.float32),
                pltpu.SemaphoreType.DMA,
                shared, shared, shared, shared, shared,
            ],
        )
        h = fn(conft, idxt, c2t, yt, cyt, ones, zeros)
        halves.append(h.reshape(5, 2, 1, 25, NBSC))
    h = jnp.concatenate(halves, axis=2)
    h = h.reshape(5, C, NBSC)
    h = jnp.pad(h, ((0, 0), (0, 0), (0, NBUCKETS - NBSC)))
    return h


BCH = 512
NBSTEP = NBUCKETS // BCH


def _post_body(h1_ref, hc_ref, hc2_ref, hy_ref, hcy_ref, minv_ref, out_ref,
               carry_ref, base1_ref,
               cqb1_ref, cqbc_ref, cqbc2_ref, cqby_ref, cqbcy_ref,
               hat1_ref, hatc_ref, hatc2_ref, haty_ref, hatcy_ref):
    j = pl.program_id(0)
    cqb_refs = (cqb1_ref, cqbc_ref, cqbc2_ref, cqby_ref, cqbcy_ref)
    hat_refs = (hat1_ref, hatc_ref, hatc2_ref, haty_ref, hatcy_ref)

    @pl.when(j == 0)
    def _():
        z16 = jnp.zeros((C, 16), jnp.float32)
        carry_ref[...] = jnp.zeros((C, 1), jnp.float32)
        base1_ref[...] = jnp.full((C, 16), -1.0, jnp.float32)
        for r in cqb_refs:
            r[...] = z16
        for r in hat_refs:
            r[...] = z16

    hs = [h1_ref[0], hc_ref[0], hc2_ref[0], hy_ref[0], hcy_ref[0]]
    h1 = hs[0]
    rr = lax.broadcasted_iota(jnp.int32, (BCH, BCH), 0)
    cc = lax.broadcasted_iota(jnp.int32, (BCH, BCH), 1)
    tri = (rr <= cc).astype(jnp.float32)
    cum_in = jax.lax.dot(h1, tri, preferred_element_type=jnp.float32)
    carry = carry_ref[...]
    cum = cum_in + carry
    prev = jnp.concatenate([carry, cum[:, :BCH - 1]], axis=1)
    carry_ref[...] = cum[:, BCH - 1:BCH]

    zero1 = jnp.zeros((C, 1), jnp.float32)
    neg1 = jnp.full((C, 1), -1.0, jnp.float32)
    b1_cols = [neg1]
    cqb_cols = [[zero1] for _ in range(5)]
    hat_cols = [[zero1] for _ in range(5)]
    for k in range(1, 15):
        t = k * (float(N) / NBINS) + 0.5
        lt = cum < t
        onehot = jnp.where((cum >= t) & (prev < t), 1.0, 0.0)
        b1_cols.append(jnp.max(jnp.where(lt, cum, -1.0), axis=1,
                               keepdims=True))
        for q in range(5):
            cqb_cols[q].append(jnp.sum(jnp.where(lt, hs[q], 0.0), axis=1,
                                       keepdims=True))
            hat_cols[q].append(jnp.sum(hs[q] * onehot, axis=1,
                                       keepdims=True))
    b1_cols.append(neg1)
    for q in range(5):
        cqb_cols[q].append(jnp.sum(hs[q], axis=1, keepdims=True))
        hat_cols[q].append(zero1)
    base1_ref[...] = jnp.maximum(base1_ref[...],
                                 jnp.concatenate(b1_cols, axis=1))
    for q in range(5):
        cqb_refs[q][...] = (cqb_refs[q][...]
                            + jnp.concatenate(cqb_cols[q], axis=1))
        hat_refs[q][...] = (hat_refs[q][...]
                            + jnp.concatenate(hat_cols[q], axis=1))

    @pl.when(j == NBSTEP - 1)
    def _():
        kk = lax.broadcasted_iota(jnp.int32, (C, 16), 1)
        tv = kk.astype(jnp.float32) * (float(N) / NBINS) + 0.5
        base1 = jnp.maximum(base1_ref[...], 0.0)
        h1b = hat_refs[0][...]
        frac = jnp.where(h1b < 0.5, 0.0,
                         (tv - base1) / jnp.maximum(h1b, 1.0))
        mn = minv_ref[...]
        cq = []
        for q in range(5):
            v = cqb_refs[q][...] + frac * hat_refs[q][...]
            cq.append(v)
        mnb = jnp.broadcast_to(mn, (C, 16))
        cq[0] = jnp.where(kk == 0, 1.0, cq[0])
        cq[1] = jnp.where(kk == 0, mnb, cq[1])
        cq[2] = jnp.where(kk == 0, mnb * mnb, cq[2])
        cq[3] = jnp.where(kk == 0, 0.0, cq[3])
        cq[4] = jnp.where(kk == 0, 0.0, cq[4])
        n_b = cq[0][:, 1:16] - cq[0][:, 0:15]
        s_c = cq[1][:, 1:16] - cq[1][:, 0:15]
        s_c2 = cq[2][:, 1:16] - cq[2][:, 0:15]
        s_y = cq[3][:, 1:16] - cq[3][:, 0:15]
        s_cy = cq[4][:, 1:16] - cq[4][:, 0:15]
        e = 1.0 / (n_b - 1.0)
        a = s_y * e
        t_b = (s_c2 - 2.0 * a * s_c + n_b * a * a
               + 2.0 * e * (s_cy - a * s_y) + e * e * s_y)
        t_b = jnp.where(n_b < 0.5, 0.0, t_b)
        out_ref[...] = jnp.sum(t_b, axis=1, keepdims=True) / float(N)


def _post_call(hist5, minv):
    s16 = pltpu.VMEM((C, 16), jnp.float32)
    return pl.pallas_call(
        _post_body,
        grid=(NBSTEP,),
        in_specs=[
            pl.BlockSpec((1, C, BCH), lambda j, q=q: (q, 0, j))
            for q in range(5)
        ] + [pl.BlockSpec((C, 1), lambda j: (0, 0))],
        out_specs=pl.BlockSpec((C, 1), lambda j: (0, 0)),
        out_shape=jax.ShapeDtypeStruct((C, 1), jnp.float32),
        scratch_shapes=[pltpu.VMEM((C, 1), jnp.float32), s16,
                        s16, s16, s16, s16, s16,
                        s16, s16, s16, s16, s16],
    )(hist5, hist5, hist5, hist5, hist5, minv)


def kernel(logits_target, labels_target):
    labels3d = labels_target.reshape(N // NBA, 1, NBA)
    conft, idxt, c2t, yt, cyt, minv = _softmax_call(logits_target, labels3d)
    hist5 = _hist_call(conft, idxt, c2t, yt, cyt)
    out = _post_call(hist5, minv)
    return out.reshape(C)


# speedup vs baseline: 1.0051x; 1.0051x over previous
import jax
import jax.numpy as jnp
from jax import lax
from jax.experimental import pallas as pl
from jax.experimental.pallas import tpu as pltpu
from jax.experimental.pallas import tpu_sc as plsc

N = 65536
C = 100
NBINS = 15
SHIFT = 17
NBUCKETS = 8192
NBSC = 8128

NBA = 1024


def _softmax_body(logits_ref, labels_ref, conft_ref, idxt_ref, c2t_ref,
                  yt_ref, cyt_ref, minv_ref):
    i = pl.program_id(0)
    x = logits_ref[...]
    m = jnp.max(x, axis=1, keepdims=True)
    e = jnp.exp(x - m)
    s = jnp.sum(e, axis=1, keepdims=True)
    conf = e / s
    confp = jnp.pad(conf, ((0, 0), (0, 128 - C)))
    conft = confp.T[0:C, :]
    conft_ref[...] = conft
    bits = lax.bitcast_convert_type(conft, jnp.int32)
    bucket = lax.shift_right_logical(bits, SHIFT)
    bucket = jnp.minimum(bucket, NBSC - 1)
    rows = lax.broadcasted_iota(jnp.int32, (C, 1), 0)
    local = jnp.mod(rows, 25)
    idxt_ref[...] = bucket + local * NBSC
    c2t_ref[...] = conft * conft
    labels = labels_ref[0]
    y = (labels == rows).astype(jnp.float32)
    yt_ref[...] = y
    cyt_ref[...] = conft * y
    mn = jnp.min(conft, axis=1, keepdims=True)

    @pl.when(i == 0)
    def _():
        minv_ref[...] = mn

    @pl.when(i > 0)
    def _():
        minv_ref[...] = jnp.minimum(minv_ref[...], mn)


def _softmax_call(logits, labels3d):
    big = jax.ShapeDtypeStruct((C, N), jnp.float32)
    return pl.pallas_call(
        _softmax_body,
        grid=(N // NBA,),
        in_specs=[
            pl.BlockSpec((NBA, C), lambda i: (i, 0)),
            pl.BlockSpec((1, 1, NBA), lambda i: (i, 0, 0)),
        ],
        out_specs=[
            pl.BlockSpec((C, NBA), lambda i: (0, i)),
            pl.BlockSpec((C, NBA), lambda i: (0, i)),
            pl.BlockSpec((C, NBA), lambda i: (0, i)),
            pl.BlockSpec((C, NBA), lambda i: (0, i)),
            pl.BlockSpec((C, NBA), lambda i: (0, i)),
            pl.BlockSpec((C, 1), lambda i: (0, 0)),
        ],
        out_shape=[
            big,
            jax.ShapeDtypeStruct((C, N), jnp.int32),
            big, big, big,
            jax.ShapeDtypeStruct((C, 1), jnp.float32),
        ],
    )(logits, labels3d)


NCORES = 2
CLS_PER_CORE = C // NCORES
CH = 8192
NCHUNK = N // CH


def _make_hist_body(base):
    def _hist_body(conf_hbm, idx_hbm, c2_hbm, y_hbm, cy_hbm, ones_hbm,
                   zeros_hbm, out_hbm, idx_v, vc_v, vc2_v, vy_v, vcy_v,
                   ones_v, stage_v, sem, h1_sh, hc_sh, hc2_sh, hy_sh,
                   hcy_sh):
        cid = lax.axis_index("c")
        sid = lax.axis_index("s")
        cnt = jnp.where(sid < 9, 2, 1)
        start_local = sid + jnp.minimum(sid, 9)
        shs = (h1_sh, hc_sh, hc2_sh, hy_sh, hcy_sh)

        pltpu.sync_copy(ones_hbm, ones_v)
        pltpu.sync_copy(zeros_hbm, stage_v)

        def do_class(z):
            local = start_local + z
            cls = cid * CLS_PER_CORE + base + local
            hbase = local * NBSC
            for sh in shs:
                pltpu.sync_copy(stage_v, sh.at[pl.ds(hbase, NBSC)])
            for ch in range(NCHUNK):
                sl = pl.ds(ch * CH, CH)
                descs = [
                    pltpu.async_copy(idx_hbm.at[cls, sl], idx_v, sem),
                    pltpu.async_copy(conf_hbm.at[cls, sl], vc_v, sem),
                    pltpu.async_copy(c2_hbm.at[cls, sl], vc2_v, sem),
                    pltpu.async_copy(y_hbm.at[cls, sl], vy_v, sem),
                    pltpu.async_copy(cy_hbm.at[cls, sl], vcy_v, sem),
                ]
                for d in descs:
                    d.wait()
                sdescs = [
                    pltpu.async_copy(val, sh.at[plsc.Indices(idx_v)],
                                     sem, add=True)
                    for sh, val in zip(shs,
                                       (ones_v, vc_v, vc2_v, vy_v, vcy_v))
                ]
                for d in sdescs:
                    d.wait()
            for q, sh in enumerate(shs):
                pltpu.sync_copy(sh.at[pl.ds(hbase, NBSC)], stage_v)
                pltpu.sync_copy(stage_v, out_hbm.at[(q * 2 + cid) * 25
                                                    + local])
            pltpu.sync_copy(zeros_hbm, stage_v)

        def loop_body(z, _):
            @pl.when(z < cnt)
            def _():
                do_class(z)
            return 0

        lax.fori_loop(0, 2, loop_body, 0)

    return _hist_body


def _hist_call(conft, idxt, c2t, yt, cyt):
    mesh = plsc.VectorSubcoreMesh(core_axis_name="c", subcore_axis_name="s")
    ones = jnp.ones((CH,), jnp.float32)
    zeros = jnp.zeros((NBSC,), jnp.float32)
    buf = pltpu.VMEM((CH,), jnp.float32)
    shared = pltpu.VMEM_SHARED((25 * NBSC,), jnp.float32)
    halves = []
    for base in (0, 25):
        fn = pl.kernel(
            _make_hist_body(base),
            out_type=jax.ShapeDtypeStruct((5 * 2 * 25, NBSC), jnp.float32),
            mesh=mesh,
            scratch_types=[
                pltpu.VMEM((CH,), jnp.int32),
                buf, buf, buf, buf,
                buf,
                pltpu.VMEM((NBSC,), jnp.float32),
                pltpu.SemaphoreType.DMA,
                shared, shared, shared, shared, shared,
            ],
        )
        h = fn(conft, idxt, c2t, yt, cyt, ones, zeros)
        halves.append(h.reshape(5, 2, 1, 25, NBSC))
    h = jnp.concatenate(halves, axis=2)
    h = h.reshape(5, C, NBSC)
    h = jnp.pad(h, ((0, 0), (0, 0), (0, NBUCKETS - NBSC)))
    return h


BCH = 512
NBSTEP = NBUCKETS // BCH


def _post_body(h1_ref, hc_ref, hc2_ref, hy_ref, hcy_ref, minv_ref, out_ref,
               carry_ref, base1_ref,
               cqb1_ref, cqbc_ref, cqbc2_ref, cqby_ref, cqbcy_ref,
               hat1_ref, hatc_ref, hatc2_ref, haty_ref, hatcy_ref):
    j = pl.program_id(0)
    cqb_refs = (cqb1_ref, cqbc_ref, cqbc2_ref, cqby_ref, cqbcy_ref)
    hat_refs = (hat1_ref, hatc_ref, hatc2_ref, haty_ref, hatcy_ref)

    @pl.when(j == 0)
    def _():
        z16 = jnp.zeros((C, 16), jnp.float32)
        carry_ref[...] = jnp.zeros((C, 1), jnp.float32)
        base1_ref[...] = jnp.full((C, 16), -1.0, jnp.float32)
        for r in cqb_refs:
            r[...] = z16
        for r in hat_refs:
            r[...] = z16

    hs = [h1_ref[0], hc_ref[0], hc2_ref[0], hy_ref[0], hcy_ref[0]]
    h1 = hs[0]
    rr = lax.broadcasted_iota(jnp.int32, (BCH, BCH), 0)
    cc = lax.broadcasted_iota(jnp.int32, (BCH, BCH), 1)
    tri = (rr <= cc).astype(jnp.float32)
    cum_in = jax.lax.dot(h1, tri, preferred_element_type=jnp.float32)
    carry = carry_ref[...]
    cum = cum_in + carry
    prev = jnp.concatenate([carry, cum[:, :BCH - 1]], axis=1)
    carry_ref[...] = cum[:, BCH - 1:BCH]

    zero1 = jnp.zeros((C, 1), jnp.float32)
    neg1 = jnp.full((C, 1), -1.0, jnp.float32)
    b1_cols = [neg1]
    cqb_cols = [[zero1] for _ in range(5)]
    hat_cols = [[zero1] for _ in range(5)]
    for k in range(1, 15):
        t = k * (float(N) / NBINS) + 0.5
        lt = cum < t
        onehot = jnp.where((cum >= t) & (prev < t), 1.0, 0.0)
        b1_cols.append(jnp.max(jnp.where(lt, cum, -1.0), axis=1,
                               keepdims=True))
        for q in range(5):
            cqb_cols[q].append(jnp.sum(jnp.where(lt, hs[q], 0.0), axis=1,
                                       keepdims=True))
            hat_cols[q].append(jnp.sum(hs[q] * onehot, axis=1,
                                       keepdims=True))
    b1_cols.append(neg1)
    for q in range(5):
        cqb_cols[q].append(jnp.sum(hs[q], axis=1, keepdims=True))
        hat_cols[q].append(zero1)
    base1_ref[...] = jnp.maximum(base1_ref[...],
                                 jnp.concatenate(b1_cols, axis=1))
    for q in range(5):
        cqb_refs[q][...] = (cqb_refs[q][...]
                            + jnp.concatenate(cqb_cols[q], axis=1))
        hat_refs[q][...] = (hat_refs[q][...]
                            + jnp.concatenate(hat_cols[q], axis=1))

    @pl.when(j == NBSTEP - 1)
    def _():
        kk = lax.broadcasted_iota(jnp.int32, (C, 16), 1)
        tv = kk.astype(jnp.float32) * (float(N) / NBINS) + 0.5
        base1 = jnp.maximum(base1_ref[...], 0.0)
        h1b = hat_refs[0][...]
        frac = jnp.where(h1b < 0.5, 0.0,
                         (tv - base1) / jnp.maximum(h1b, 1.0))
        mn = minv_ref[...]
        cq = []
        for q in range(5):
            v = cqb_refs[q][...] + frac * hat_refs[q][...]
            cq.append(v)
        mnb = jnp.broadcast_to(mn, (C, 16))
        cq[0] = jnp.where(kk == 0, 1.0, cq[0])
        cq[1] = jnp.where(kk == 0, mnb, cq[1])
        cq[2] = jnp.where(kk == 0, mnb * mnb, cq[2])
        cq[3] = jnp.where(kk == 0, 0.0, cq[3])
        cq[4] = jnp.where(kk == 0, 0.0, cq[4])
        n_b = cq[0][:, 1:16] - cq[0][:, 0:15]
        s_c = cq[1][:, 1:16] - cq[1][:, 0:15]
        s_c2 = cq[2][:, 1:16] - cq[2][:, 0:15]
        s_y = cq[3][:, 1:16] - cq[3][:, 0:15]
        s_cy = cq[4][:, 1:16] - cq[4][:, 0:15]
        e = 1.0 / (n_b - 1.0)
        a = s_y * e
        t_b = (s_c2 - 2.0 * a * s_c + n_b * a * a
               + 2.0 * e * (s_cy - a * s_y) + e * e * s_y)
        t_b = jnp.where(n_b < 0.5, 0.0, t_b)
        out_ref[...] = jnp.sum(t_b, axis=1, keepdims=True) / float(N)


def _post_call(hist5, minv):
    s16 = pltpu.VMEM((C, 16), jnp.float32)
    return pl.pallas_call(
        _post_body,
        grid=(NBSTEP,),
        in_specs=[
            pl.BlockSpec((1, C, BCH), lambda j, q=q: (q, 0, j))
            for q in range(5)
        ] + [pl.BlockSpec((C, 1), lambda j: (0, 0))],
        out_specs=pl.BlockSpec((C, 1), lambda j: (0, 0)),
        out_shape=jax.ShapeDtypeStruct((C, 1), jnp.float32),
        scratch_shapes=[pltpu.VMEM((C, 1), jnp.float32), s16,
                        s16, s16, s16, s16, s16,
                        s16, s16, s16, s16, s16],
    )(hist5, hist5, hist5, hist5, hist5, minv)


def kernel(logits_target, labels_target):
    labels3d = labels_target.reshape(N // NBA, 1, NBA)
    conft, idxt, c2t, yt, cyt, minv = _softmax_call(logits_target, labels3d)
    hist5 = _hist_call(conft, idxt, c2t, yt, cyt)
    out = _post_call(hist5, minv)
    return out.reshape(C)


# speedup vs baseline: 1.4518x; 1.4444x over previous
import jax
import jax.numpy as jnp
from jax import lax
from jax.experimental import pallas as pl
from jax.experimental.pallas import tpu as pltpu
from jax.experimental.pallas import tpu_sc as plsc

N = 65536
C = 100
NBINS = 15
SHIFT = 16
NBUCKETS = 16384

NBA = 1024


def _softmax_body(logits_ref, conft_ref, idxt_ref, minv_ref, maxv_ref):
    i = pl.program_id(0)
    x = logits_ref[...]
    m = jnp.max(x, axis=1, keepdims=True)
    e = jnp.exp(x - m)
    s = jnp.sum(e, axis=1, keepdims=True)
    conf = e / s
    confp = jnp.pad(conf, ((0, 0), (0, 128 - C)))
    conft = confp.T[0:C, :]
    conft_ref[...] = conft
    bits = lax.bitcast_convert_type(conft, jnp.int32)
    bucket = lax.shift_right_logical(bits, SHIFT)
    rows = lax.broadcasted_iota(jnp.int32, (C, 1), 0)
    local = jnp.where(rows >= C // 2, rows - C // 2, rows)
    idxt_ref[...] = bucket + local * NBUCKETS
    mn = jnp.min(conft, axis=1, keepdims=True)
    mx = jnp.max(conft, axis=1, keepdims=True)

    @pl.when(i == 0)
    def _():
        minv_ref[...] = mn
        maxv_ref[...] = mx

    @pl.when(i > 0)
    def _():
        minv_ref[...] = jnp.minimum(minv_ref[...], mn)
        maxv_ref[...] = jnp.maximum(maxv_ref[...], mx)


def _softmax_call(logits):
    return pl.pallas_call(
        _softmax_body,
        grid=(N // NBA,),
        in_specs=[pl.BlockSpec((NBA, C), lambda i: (i, 0))],
        out_specs=[
            pl.BlockSpec((C, NBA), lambda i: (0, i)),
            pl.BlockSpec((C, NBA), lambda i: (0, i)),
            pl.BlockSpec((C, 1), lambda i: (0, 0)),
            pl.BlockSpec((C, 1), lambda i: (0, 0)),
        ],
        out_shape=[
            jax.ShapeDtypeStruct((C, N), jnp.float32),
            jax.ShapeDtypeStruct((C, N), jnp.int32),
            jax.ShapeDtypeStruct((C, 1), jnp.float32),
            jax.ShapeDtypeStruct((C, 1), jnp.float32),
        ],
    )(logits)


NCORES = 2
NSUB = 16
CLS_PER_CORE = C // NCORES
CH = 16384
NCHUNK = N // CH
ROWS = CH // 128


def _hist_body(idx2_hbm, ones_hbm, zeros_hbm, out_hbm,
               idx_v, ones_v, sem, hist_sh):
    cid = lax.axis_index("c")
    sid = lax.axis_index("s")
    cnt = jnp.where(sid < 2, 4, 3)
    start_local = 3 * sid + jnp.minimum(sid, 2)
    core_base = cid * CLS_PER_CORE

    pltpu.sync_copy(ones_hbm, ones_v)

    def do_class(z):
        local = start_local + z
        cls = core_base + local
        hbase = local * NBUCKETS
        pltpu.sync_copy(zeros_hbm, hist_sh.at[pl.ds(hbase, NBUCKETS)])
        for ch in range(NCHUNK):
            pltpu.sync_copy(idx2_hbm.at[cls, pl.ds(ch * CH, CH)], idx_v)
            pltpu.sync_copy(ones_v, hist_sh.at[plsc.Indices(idx_v)],
                            add=True)
        pltpu.sync_copy(hist_sh.at[pl.ds(hbase, NBUCKETS)], out_hbm.at[cls])

    def loop_body(z, _):
        @pl.when(z < cnt)
        def _():
            do_class(z)
        return 0

    lax.fori_loop(0, 4, loop_body, 0)


def _hist_call(idxt):
    mesh = plsc.VectorSubcoreMesh(core_axis_name="c", subcore_axis_name="s")
    ones = jnp.ones((CH,), jnp.float32)
    zeros = jnp.zeros((NBUCKETS,), jnp.float32)
    fn = pl.kernel(
        _hist_body,
        out_type=jax.ShapeDtypeStruct((C, NBUCKETS), jnp.float32),
        mesh=mesh,
        scratch_types=[
            pltpu.VMEM((CH,), jnp.int32),
            pltpu.VMEM((CH,), jnp.float32),
            pltpu.SemaphoreType.DMA,
            pltpu.VMEM_SHARED((CLS_PER_CORE * NBUCKETS,), jnp.float32),
        ],
    )
    return fn(idxt, ones, zeros)


BCH = 512
NBSTEP = NBUCKETS // BCH
BIG = 3.0e38


def _edges_body(hist_ref, minv_ref, maxv_ref, edges_ref,
                carry_ref, bacc_ref, cumlo_ref, cumhi_ref):
    j = pl.program_id(0)

    @pl.when(j == 0)
    def _():
        carry_ref[...] = jnp.zeros((C, 1), jnp.float32)
        bacc_ref[...] = jnp.zeros((C, 16), jnp.float32)
        cumlo_ref[...] = jnp.full((C, 16), -1.0, jnp.float32)
        cumhi_ref[...] = jnp.full((C, 16), BIG, jnp.float32)

    h = hist_ref[...]
    rr = lax.broadcasted_iota(jnp.int32, (BCH, BCH), 0)
    cc = lax.broadcasted_iota(jnp.int32, (BCH, BCH), 1)
    tri = (rr <= cc).astype(jnp.float32)
    cum_in = jax.lax.dot(h, tri, preferred_element_type=jnp.float32)
    cum = cum_in + carry_ref[...]
    carry_ref[...] = cum[:, BCH - 1:BCH]

    zero1 = jnp.zeros((C, 1), jnp.float32)
    neg1 = jnp.full((C, 1), -1.0, jnp.float32)
    big1 = jnp.full((C, 1), BIG, jnp.float32)
    nlt_cols, mlo_cols, mhi_cols = [zero1], [neg1], [big1]
    for k in range(1, 15):
        t = k * (float(N) / NBINS) + 0.5
        lt = cum < t
        nlt_cols.append(jnp.sum(jnp.where(lt, 1.0, 0.0), axis=1,
                                keepdims=True))
        mlo_cols.append(jnp.max(jnp.where(lt, cum, -1.0), axis=1,
                                keepdims=True))
        mhi_cols.append(jnp.min(jnp.where(lt, BIG, cum), axis=1,
                                keepdims=True))
    nlt_cols.append(zero1)
    mlo_cols.append(neg1)
    mhi_cols.append(big1)
    bacc_ref[...] = bacc_ref[...] + jnp.concatenate(nlt_cols, axis=1)
    cumlo_ref[...] = jnp.maximum(cumlo_ref[...],
                                 jnp.concatenate(mlo_cols, axis=1))
    cumhi_ref[...] = jnp.minimum(cumhi_ref[...],
                                 jnp.concatenate(mhi_cols, axis=1))

    @pl.when(j == NBSTEP - 1)
    def _():
        b_i = bacc_ref[...].astype(jnp.int32)
        vlo = lax.bitcast_convert_type(b_i << SHIFT, jnp.float32)
        vhi = lax.bitcast_convert_type((b_i + 1) << SHIFT, jnp.float32)
        lo = jnp.maximum(cumlo_ref[...], 0.0)
        hi = cumhi_ref[...]
        kk = lax.broadcasted_iota(jnp.int32, (C, 16), 1).astype(jnp.float32)
        pos = kk * (float(N) / NBINS) + 0.5
        frac = (pos - lo) / jnp.maximum(hi - lo, 1.0)
        edge = vlo + frac * (vhi - vlo)
        ki = lax.broadcasted_iota(jnp.int32, (C, 16), 1)
        mn = jnp.broadcast_to(minv_ref[...], (C, 16))
        mx = jnp.broadcast_to(maxv_ref[...], (C, 16))
        edges_ref[...] = jnp.where(ki == 0, mn,
                                   jnp.where(ki == 15, mx, edge))


def _edges_call(hist, minv, maxv):
    return pl.pallas_call(
        _edges_body,
        grid=(NBSTEP,),
        in_specs=[
            pl.BlockSpec((C, BCH), lambda j: (0, j)),
            pl.BlockSpec((C, 1), lambda j: (0, 0)),
            pl.BlockSpec((C, 1), lambda j: (0, 0)),
        ],
        out_specs=pl.BlockSpec((C, 16), lambda j: (0, 0)),
        out_shape=jax.ShapeDtypeStruct((C, 16), jnp.float32),
        scratch_shapes=[
            pltpu.VMEM((C, 1), jnp.float32),
            pltpu.VMEM((C, 16), jnp.float32),
            pltpu.VMEM((C, 16), jnp.float32),
            pltpu.VMEM((C, 16), jnp.float32),
        ],
    )(hist, minv, maxv)


ND = 2048
NDSTEP = N // ND


def _bin_body(conft_ref, labels_ref, edges_ref, out_ref,
              u1_ref, uc_ref, uc2_ref, uy_ref, ucy_ref):
    i = pl.program_id(0)

    @pl.when(i == 0)
    def _():
        z = jnp.zeros((C, 16), jnp.float32)
        u1_ref[...] = z
        uc_ref[...] = z
        uc2_ref[...] = z
        uy_ref[...] = z
        ucy_ref[...] = z

    conf = conft_ref[...]
    labels = labels_ref[0]
    clsid = lax.broadcasted_iota(jnp.int32, (C, ND), 0)
    y = (labels == clsid).astype(jnp.float32)
    c2 = conf * conf
    cy = conf * y
    edges = edges_ref[...]

    cols = []
    for k in range(16):
        m = conf > edges[:, k:k + 1]
        s1 = jnp.sum(jnp.where(m, 1.0, 0.0), axis=1, keepdims=True)
        sc = jnp.sum(jnp.where(m, conf, 0.0), axis=1, keepdims=True)
        sc2 = jnp.sum(jnp.where(m, c2, 0.0), axis=1, keepdims=True)
        sy = jnp.sum(jnp.where(m, y, 0.0), axis=1, keepdims=True)
        scy = jnp.sum(jnp.where(m, cy, 0.0), axis=1, keepdims=True)
        cols.append((s1, sc, sc2, sy, scy))
    u1_ref[...] = u1_ref[...] + jnp.concatenate([c[0] for c in cols], axis=1)
    uc_ref[...] = uc_ref[...] + jnp.concatenate([c[1] for c in cols], axis=1)
    uc2_ref[...] = uc2_ref[...] + jnp.concatenate([c[2] for c in cols],
                                                  axis=1)
    uy_ref[...] = uy_ref[...] + jnp.concatenate([c[3] for c in cols], axis=1)
    ucy_ref[...] = ucy_ref[...] + jnp.concatenate([c[4] for c in cols],
                                                  axis=1)

    @pl.when(i == NDSTEP - 1)
    def _():
        u1 = u1_ref[...]
        uc = uc_ref[...]
        uc2 = uc2_ref[...]
        uy = uy_ref[...]
        ucy = ucy_ref[...]
        n_b = u1[:, 0:15] - u1[:, 1:16]
        s_c = uc[:, 0:15] - uc[:, 1:16]
        s_c2 = uc2[:, 0:15] - uc2[:, 1:16]
        s_y = uy[:, 0:15] - uy[:, 1:16]
        s_cy = ucy[:, 0:15] - ucy[:, 1:16]
        e = 1.0 / (n_b - 1.0)
        a = s_y * e
        t = (s_c2 - 2.0 * a * s_c + n_b * a * a
             + 2.0 * e * (s_cy - a * s_y) + e * e * s_y)
        t = jnp.where(n_b < 0.5, 0.0, t)
        out_ref[...] = jnp.sum(t, axis=1, keepdims=True) / float(N)


def _bin_call(conft, labels3d, edges):
    return pl.pallas_call(
        _bin_body,
        grid=(NDSTEP,),
        in_specs=[
            pl.BlockSpec((C, ND), lambda i: (0, i)),
            pl.BlockSpec((1, 1, ND), lambda i: (i, 0, 0)),
            pl.BlockSpec((C, 16), lambda i: (0, 0)),
        ],
        out_specs=pl.BlockSpec((C, 1), lambda i: (0, 0)),
        out_shape=jax.ShapeDtypeStruct((C, 1), jnp.float32),
        scratch_shapes=[
            pltpu.VMEM((C, 16), jnp.float32),
            pltpu.VMEM((C, 16), jnp.float32),
            pltpu.VMEM((C, 16), jnp.float32),
            pltpu.VMEM((C, 16), jnp.float32),
            pltpu.VMEM((C, 16), jnp.float32),
        ],
    )(conft, labels3d, edges)


def kernel(logits_target, labels_target):
    conft, idxt, minv, maxv = _softmax_call(logits_target)
    hist = _hist_call(idxt)
    edges = _edges_call(hist, minv, maxv)
    labels3d = labels_target.reshape(NDSTEP, 1, ND)
    out = _bin_call(conft, labels3d, edges)
    return out.reshape(C)


# speedup vs baseline: 1.7912x; 1.2338x over previous
import jax
import jax.numpy as jnp
from jax import lax
from jax.experimental import pallas as pl
from jax.experimental.pallas import tpu as pltpu
from jax.experimental.pallas import tpu_sc as plsc

N = 65536
C = 100
NBINS = 15
SHIFT = 16
NBUCKETS = 16384

NBA = 1024


def _softmax_body(logits_ref, conft_ref, idxt_ref, minv_ref, maxv_ref):
    i = pl.program_id(0)
    x = logits_ref[...]
    m = jnp.max(x, axis=1, keepdims=True)
    e = jnp.exp(x - m)
    s = jnp.sum(e, axis=1, keepdims=True)
    conf = e / s
    confp = jnp.pad(conf, ((0, 0), (0, 128 - C)))
    conft = confp.T[0:C, :]
    conft_ref[...] = conft
    bits = lax.bitcast_convert_type(conft, jnp.int32)
    bucket = lax.shift_right_logical(bits, SHIFT)
    rows = lax.broadcasted_iota(jnp.int32, (C, 1), 0)
    local = jnp.where(rows >= C // 2, rows - C // 2, rows)
    idxt_ref[...] = bucket + local * NBUCKETS
    mn = jnp.min(conft, axis=1, keepdims=True)
    mx = jnp.max(conft, axis=1, keepdims=True)

    @pl.when(i == 0)
    def _():
        minv_ref[...] = mn
        maxv_ref[...] = mx

    @pl.when(i > 0)
    def _():
        minv_ref[...] = jnp.minimum(minv_ref[...], mn)
        maxv_ref[...] = jnp.maximum(maxv_ref[...], mx)


def _softmax_call(logits):
    return pl.pallas_call(
        _softmax_body,
        grid=(N // NBA,),
        in_specs=[pl.BlockSpec((NBA, C), lambda i: (i, 0))],
        out_specs=[
            pl.BlockSpec((C, NBA), lambda i: (0, i)),
            pl.BlockSpec((C, NBA), lambda i: (0, i)),
            pl.BlockSpec((C, 1), lambda i: (0, 0)),
            pl.BlockSpec((C, 1), lambda i: (0, 0)),
        ],
        out_shape=[
            jax.ShapeDtypeStruct((C, N), jnp.float32),
            jax.ShapeDtypeStruct((C, N), jnp.int32),
            jax.ShapeDtypeStruct((C, 1), jnp.float32),
            jax.ShapeDtypeStruct((C, 1), jnp.float32),
        ],
    )(logits)


NCORES = 2
NSUB = 16
CLS_PER_CORE = C // NCORES
CH = 16384
NCHUNK = N // CH
ROWS = CH // 128


def _hist_body(idx2_hbm, ones_hbm, zeros_hbm, out_hbm,
               idx_v, ones_v, sem, hist_sh):
    cid = lax.axis_index("c")
    sid = lax.axis_index("s")
    cnt = jnp.where(sid < 2, 4, 3)
    start_local = 3 * sid + jnp.minimum(sid, 2)
    core_base = cid * CLS_PER_CORE

    pltpu.sync_copy(ones_hbm, ones_v)

    def do_class(z):
        local = start_local + z
        cls = core_base + local
        hbase = local * NBUCKETS
        pltpu.sync_copy(zeros_hbm, hist_sh.at[pl.ds(hbase, NBUCKETS)])
        for ch in range(NCHUNK):
            pltpu.sync_copy(idx2_hbm.at[cls, pl.ds(ch * CH, CH)], idx_v)
            pltpu.sync_copy(ones_v, hist_sh.at[plsc.Indices(idx_v)],
                            add=True)
        pltpu.sync_copy(hist_sh.at[pl.ds(hbase, NBUCKETS)], out_hbm.at[cls])

    def loop_body(z, _):
        @pl.when(z < cnt)
        def _():
            do_class(z)
        return 0

    lax.fori_loop(0, 4, loop_body, 0)


def _hist_call(idxt):
    mesh = plsc.VectorSubcoreMesh(core_axis_name="c", subcore_axis_name="s")
    ones = jnp.ones((CH,), jnp.float32)
    zeros = jnp.zeros((NBUCKETS,), jnp.float32)
    fn = pl.kernel(
        _hist_body,
        out_type=jax.ShapeDtypeStruct((C, NBUCKETS), jnp.float32),
        mesh=mesh,
        scratch_types=[
            pltpu.VMEM((CH,), jnp.int32),
            pltpu.VMEM((CH,), jnp.float32),
            pltpu.SemaphoreType.DMA,
            pltpu.VMEM_SHARED((CLS_PER_CORE * NBUCKETS,), jnp.float32),
        ],
    )
    return fn(idxt, ones, zeros)


BCH = 512
NBSTEP = NBUCKETS // BCH
BIG = 3.0e38


def _edges_body(hist_ref, minv_ref, maxv_ref, edges_ref,
                carry_ref, bacc_ref, cumlo_ref, cumhi_ref):
    j = pl.program_id(0)

    @pl.when(j == 0)
    def _():
        carry_ref[...] = jnp.zeros((C, 1), jnp.float32)
        bacc_ref[...] = jnp.zeros((C, 16), jnp.float32)
        cumlo_ref[...] = jnp.full((C, 16), -1.0, jnp.float32)
        cumhi_ref[...] = jnp.full((C, 16), BIG, jnp.float32)

    h = hist_ref[...]
    rr = lax.broadcasted_iota(jnp.int32, (BCH, BCH), 0)
    cc = lax.broadcasted_iota(jnp.int32, (BCH, BCH), 1)
    tri = (rr <= cc).astype(jnp.float32)
    cum_in = jax.lax.dot(h, tri, preferred_element_type=jnp.float32)
    cum = cum_in + carry_ref[...]
    carry_ref[...] = cum[:, BCH - 1:BCH]

    zero1 = jnp.zeros((C, 1), jnp.float32)
    neg1 = jnp.full((C, 1), -1.0, jnp.float32)
    big1 = jnp.full((C, 1), BIG, jnp.float32)
    nlt_cols, mlo_cols, mhi_cols = [zero1], [neg1], [big1]
    for k in range(1, 15):
        t = k * (float(N) / NBINS) + 0.5
        lt = cum < t
        nlt_cols.append(jnp.sum(jnp.where(lt, 1.0, 0.0), axis=1,
                                keepdims=True))
        mlo_cols.append(jnp.max(jnp.where(lt, cum, -1.0), axis=1,
                                keepdims=True))
        mhi_cols.append(jnp.min(jnp.where(lt, BIG, cum), axis=1,
                                keepdims=True))
    nlt_cols.append(zero1)
    mlo_cols.append(neg1)
    mhi_cols.append(big1)
    bacc_ref[...] = bacc_ref[...] + jnp.concatenate(nlt_cols, axis=1)
    cumlo_ref[...] = jnp.maximum(cumlo_ref[...],
                                 jnp.concatenate(mlo_cols, axis=1))
    cumhi_ref[...] = jnp.minimum(cumhi_ref[...],
                                 jnp.concatenate(mhi_cols, axis=1))

    @pl.when(j == NBSTEP - 1)
    def _():
        b_i = bacc_ref[...].astype(jnp.int32)
        vlo = lax.bitcast_convert_type(b_i << SHIFT, jnp.float32)
        vhi = lax.bitcast_convert_type((b_i + 1) << SHIFT, jnp.float32)
        lo = jnp.maximum(cumlo_ref[...], 0.0)
        hi = cumhi_ref[...]
        kk = lax.broadcasted_iota(jnp.int32, (C, 16), 1).astype(jnp.float32)
        pos = kk * (float(N) / NBINS) + 0.5
        frac = (pos - lo) / jnp.maximum(hi - lo, 1.0)
        edge = vlo + frac * (vhi - vlo)
        ki = lax.broadcasted_iota(jnp.int32, (C, 16), 1)
        mn = jnp.broadcast_to(minv_ref[...], (C, 16))
        mx = jnp.broadcast_to(maxv_ref[...], (C, 16))
        edges_ref[...] = jnp.where(ki == 0, mn,
                                   jnp.where(ki == 15, mx, edge))


def _edges_call(hist, minv, maxv):
    return pl.pallas_call(
        _edges_body,
        grid=(NBSTEP,),
        in_specs=[
            pl.BlockSpec((C, BCH), lambda j: (0, j)),
            pl.BlockSpec((C, 1), lambda j: (0, 0)),
            pl.BlockSpec((C, 1), lambda j: (0, 0)),
        ],
        out_specs=pl.BlockSpec((C, 16), lambda j: (0, 0)),
        out_shape=jax.ShapeDtypeStruct((C, 16), jnp.float32),
        scratch_shapes=[
            pltpu.VMEM((C, 1), jnp.float32),
            pltpu.VMEM((C, 16), jnp.float32),
            pltpu.VMEM((C, 16), jnp.float32),
            pltpu.VMEM((C, 16), jnp.float32),
        ],
    )(hist, minv, maxv)


ND = 2048
NDSTEP = N // ND


def _bin_body(conft_ref, labels_ref, edges_ref, out_ref,
              u1_ref, uc_ref, uc2_ref, uy_ref, ucy_ref):
    i = pl.program_id(0)

    @pl.when(i == 0)
    def _():
        z = jnp.zeros((C, 16), jnp.float32)
        u1_ref[...] = z
        uc_ref[...] = z
        uc2_ref[...] = z
        uy_ref[...] = z
        ucy_ref[...] = z

    conf = conft_ref[...]
    labels = labels_ref[0]
    clsid = lax.broadcasted_iota(jnp.int32, (C, ND), 0)
    y = (labels == clsid).astype(jnp.float32)
    c2 = conf * conf
    cy = conf * y
    edges = edges_ref[...]

    clab = jnp.sum(cy, axis=0, keepdims=True)
    eg = lax.dot_general(edges, y, (((0,), (0,)), ((), ())),
                         preferred_element_type=jnp.float32)
    m2 = (clab > eg).astype(jnp.float32)
    uy_ref[...] = uy_ref[...] + lax.dot_general(
        y, m2, (((1,), (1,)), ((), ())),
        preferred_element_type=jnp.float32)
    ucy_ref[...] = ucy_ref[...] + lax.dot_general(
        cy, m2, (((1,), (1,)), ((), ())),
        preferred_element_type=jnp.float32)

    cols = []
    for k in range(16):
        m = conf > edges[:, k:k + 1]
        s1 = jnp.sum(jnp.where(m, 1.0, 0.0), axis=1, keepdims=True)
        sc = jnp.sum(jnp.where(m, conf, 0.0), axis=1, keepdims=True)
        sc2 = jnp.sum(jnp.where(m, c2, 0.0), axis=1, keepdims=True)
        cols.append((s1, sc, sc2))
    u1_ref[...] = u1_ref[...] + jnp.concatenate([c[0] for c in cols], axis=1)
    uc_ref[...] = uc_ref[...] + jnp.concatenate([c[1] for c in cols], axis=1)
    uc2_ref[...] = uc2_ref[...] + jnp.concatenate([c[2] for c in cols],
                                                  axis=1)

    @pl.when(i == NDSTEP - 1)
    def _():
        u1 = u1_ref[...]
        uc = uc_ref[...]
        uc2 = uc2_ref[...]
        uy = uy_ref[...]
        ucy = ucy_ref[...]
        n_b = u1[:, 0:15] - u1[:, 1:16]
        s_c = uc[:, 0:15] - uc[:, 1:16]
        s_c2 = uc2[:, 0:15] - uc2[:, 1:16]
        s_y = uy[:, 0:15] - uy[:, 1:16]
        s_cy = ucy[:, 0:15] - ucy[:, 1:16]
        e = 1.0 / (n_b - 1.0)
        a = s_y * e
        t = (s_c2 - 2.0 * a * s_c + n_b * a * a
             + 2.0 * e * (s_cy - a * s_y) + e * e * s_y)
        t = jnp.where(n_b < 0.5, 0.0, t)
        out_ref[...] = jnp.sum(t, axis=1, keepdims=True) / float(N)


def _bin_call(conft, labels3d, edges):
    return pl.pallas_call(
        _bin_body,
        grid=(NDSTEP,),
        in_specs=[
            pl.BlockSpec((C, ND), lambda i: (0, i)),
            pl.BlockSpec((1, 1, ND), lambda i: (i, 0, 0)),
            pl.BlockSpec((C, 16), lambda i: (0, 0)),
        ],
        out_specs=pl.BlockSpec((C, 1), lambda i: (0, 0)),
        out_shape=jax.ShapeDtypeStruct((C, 1), jnp.float32),
        scratch_shapes=[
            pltpu.VMEM((C, 16), jnp.float32),
            pltpu.VMEM((C, 16), jnp.float32),
            pltpu.VMEM((C, 16), jnp.float32),
            pltpu.VMEM((C, 16), jnp.float32),
            pltpu.VMEM((C, 16), jnp.float32),
        ],
    )(conft, labels3d, edges)


def kernel(logits_target, labels_target):
    conft, idxt, minv, maxv = _softmax_call(logits_target)
    hist = _hist_call(idxt)
    edges = _edges_call(hist, minv, maxv)
    labels3d = labels_target.reshape(NDSTEP, 1, ND)
    out = _bin_call(conft, labels3d, edges)
    return out.reshape(C)
